# baseline (speedup 1.0000x reference)
import functools

import jax
import jax.numpy as jnp
from jax import lax
from jax.experimental import pallas as pl
from jax.experimental.pallas import tpu as pltpu
from jax.experimental.pallas import tpu_sc as plsc

TID_DIM = 32
N_TABLE = 288
N_TOTAL = 64 * 12 * 5000
NW = 32
PER_W = N_TOTAL // NW
CHUNK = 1200
NCHUNK = PER_W // CHUNK
NGRP = CHUNK // 16

_mesh = plsc.VectorSubcoreMesh(core_axis_name="c", subcore_axis_name="s")


@functools.partial(
    pl.kernel,
    mesh=_mesh,
    out_type=jax.ShapeDtypeStruct((N_TOTAL * TID_DIM,), jnp.float32),
    compiler_params=pltpu.CompilerParams(needs_layout_passes=False),
    scratch_types=[
        pltpu.VMEM((N_TABLE * TID_DIM,), jnp.float32),
        pltpu.VMEM((CHUNK,), jnp.int32),
        pltpu.VMEM((CHUNK * TID_DIM,), jnp.float32),
        pltpu.VMEM((CHUNK * TID_DIM,), jnp.float32),
        pltpu.SemaphoreType.DMA,
        pltpu.SemaphoreType.DMA,
    ],
)
def _gather_kernel(table_hbm, idx_hbm, out_hbm, table_v, idx_v,
                   rows0_v, rows1_v, sem0, sem1):
    wid = lax.axis_index("s") * 2 + lax.axis_index("c")
    w_base = wid * PER_W
    rows_bufs = (rows0_v, rows1_v)
    sems = (sem0, sem1)

    pltpu.sync_copy(table_hbm, table_v)

    lane = lax.iota(jnp.int32, 16)
    out_lane = lane * TID_DIM

    def chunk_body(g, carry):
        for b in range(2):
            j = g * 2 + b
            base = w_base + j * CHUNK

            @pl.when(g > 0)
            def _wait_prev_store():
                pltpu.make_async_copy(
                    rows_bufs[b],
                    out_hbm.at[pl.ds(0, CHUNK * TID_DIM)],
                    sems[b],
                ).wait()

            pltpu.sync_copy(idx_hbm.at[pl.ds(base, CHUNK)], idx_v)

            def grp_body(grp, c2):
                idxv = idx_v[pl.ds(grp * 16, 16)]
                src = idxv * TID_DIM
                dst = out_lane + grp * (16 * TID_DIM)
                for d in range(TID_DIM):
                    val = plsc.load_gather(table_v, [src + d])
                    plsc.store_scatter(rows_bufs[b], [dst + d], val)
                return c2

            lax.fori_loop(0, NGRP, grp_body, 0, unroll=False)

            pltpu.async_copy(
                rows_bufs[b],
                out_hbm.at[pl.ds(base * TID_DIM, CHUNK * TID_DIM)],
                sems[b],
            )
        return carry

    lax.fori_loop(0, NCHUNK // 2, chunk_body, 0)

    for b in range(2):
        pltpu.make_async_copy(
            rows_bufs[b],
            out_hbm.at[pl.ds(0, CHUNK * TID_DIM)],
            sems[b],
        ).wait()


def kernel(tid, node_emb_in, node_emb_out, tid_table, adp_emb):
    idx = tid.reshape(-1).astype(jnp.int32)
    flat = _gather_kernel(tid_table.reshape(-1), idx)
    x_tid = flat.reshape(tid.shape + (TID_DIM,))
    return (node_emb_in, node_emb_out, x_tid, adp_emb)


# speedup vs baseline: 1.2235x; 1.2235x over previous
import functools

import jax
import jax.numpy as jnp
from jax import lax
from jax.experimental import pallas as pl
from jax.experimental.pallas import tpu as pltpu
from jax.experimental.pallas import tpu_sc as plsc

TID_DIM = 32
N_TABLE = 288
N_TOTAL = 64 * 12 * 5000
NW = 32
PER_W = N_TOTAL // NW
CHUNK = 1200
NCHUNK = PER_W // CHUNK
NGRP = CHUNK // 16

_mesh = plsc.VectorSubcoreMesh(core_axis_name="c", subcore_axis_name="s")


@functools.partial(
    pl.kernel,
    mesh=_mesh,
    out_type=jax.ShapeDtypeStruct((N_TOTAL * TID_DIM,), jnp.float32),
    compiler_params=pltpu.CompilerParams(needs_layout_passes=False),
    scratch_types=[
        pltpu.VMEM((N_TABLE * TID_DIM,), jnp.float32),
        pltpu.VMEM((CHUNK,), jnp.int32),
        pltpu.VMEM((CHUNK * TID_DIM,), jnp.float32),
        pltpu.VMEM((CHUNK * TID_DIM,), jnp.float32),
        pltpu.SemaphoreType.DMA,
        pltpu.SemaphoreType.DMA,
    ],
)
def _gather_kernel(table_hbm, idx_hbm, out_hbm, table_v, idx_v,
                   rows0_v, rows1_v, sem0, sem1):
    wid = lax.axis_index("s") * 2 + lax.axis_index("c")
    w_base = wid * PER_W
    rows_bufs = (rows0_v, rows1_v)
    sems = (sem0, sem1)

    pltpu.sync_copy(table_hbm, table_v)

    lane = lax.iota(jnp.int32, 16)
    out_lane = lane * TID_DIM

    def chunk_body(g, carry):
        for b in range(2):
            j = g * 2 + b
            base = w_base + j * CHUNK

            @pl.when(g > 0)
            def _wait_prev_store():
                pltpu.make_async_copy(
                    rows_bufs[b],
                    out_hbm.at[pl.ds(0, CHUNK * TID_DIM)],
                    sems[b],
                ).wait()

            pltpu.sync_copy(idx_hbm.at[pl.ds(base, CHUNK)], idx_v)

            @plsc.parallel_loop(0, NGRP, unroll=4)
            def grp_body(grp):
                idxv = idx_v[pl.ds(grp * 16, 16)]
                src = idxv * TID_DIM
                dst = out_lane + grp * (16 * TID_DIM)
                for d in range(TID_DIM):
                    val = plsc.load_gather(table_v, [src + d])
                    plsc.store_scatter(rows_bufs[b], [dst + d], val)

            pltpu.async_copy(
                rows_bufs[b],
                out_hbm.at[pl.ds(base * TID_DIM, CHUNK * TID_DIM)],
                sems[b],
            )
        return carry

    lax.fori_loop(0, NCHUNK // 2, chunk_body, 0)

    for b in range(2):
        pltpu.make_async_copy(
            rows_bufs[b],
            out_hbm.at[pl.ds(0, CHUNK * TID_DIM)],
            sems[b],
        ).wait()


def kernel(tid, node_emb_in, node_emb_out, tid_table, adp_emb):
    idx = tid.reshape(-1).astype(jnp.int32)
    flat = _gather_kernel(tid_table.reshape(-1), idx)
    x_tid = flat.reshape(tid.shape + (TID_DIM,))
    return (node_emb_in, node_emb_out, x_tid, adp_emb)


# speedup vs baseline: 1.9737x; 1.6132x over previous
import functools

import jax
import jax.numpy as jnp
from jax import lax
from jax.experimental import pallas as pl
from jax.experimental.pallas import tpu as pltpu
from jax.experimental.pallas import tpu_sc as plsc

TID_DIM = 32
N_TABLE = 288
N_TOTAL = 64 * 12 * 5000
NW = 32
PER_W = N_TOTAL // NW
CHUNK = 1200
DCH = 800
TCH = CHUNK - DCH
NCHUNK = PER_W // CHUNK
NGRP = TCH // 16

_mesh = plsc.VectorSubcoreMesh(core_axis_name="c", subcore_axis_name="s")


@functools.partial(
    pl.kernel,
    mesh=_mesh,
    out_type=jax.ShapeDtypeStruct((N_TOTAL, TID_DIM), jnp.float32),
    compiler_params=pltpu.CompilerParams(
        use_tc_tiling_on_sc=False, needs_layout_passes=False
    ),
    scratch_types=[
        pltpu.VMEM((N_TABLE, TID_DIM), jnp.float32),
        pltpu.VMEM((CHUNK,), jnp.int32),
        pltpu.VMEM((CHUNK,), jnp.int32),
        pltpu.VMEM((DCH, TID_DIM), jnp.float32),
        pltpu.VMEM((DCH, TID_DIM), jnp.float32),
        pltpu.VMEM((TCH, TID_DIM), jnp.float32),
        pltpu.VMEM((TCH, TID_DIM), jnp.float32),
        pltpu.SemaphoreType.DMA,
        pltpu.SemaphoreType.DMA,
        pltpu.SemaphoreType.DMA,
        pltpu.SemaphoreType.DMA,
        pltpu.SemaphoreType.DMA,
    ],
)
def _gather_kernel(table2d_hbm, idx_hbm, out_hbm,
                   table_v, idx0_v, idx1_v, drows0_v, drows1_v,
                   trows0_v, trows1_v, dsem0, dsem1, tsem0, tsem1, gsem):
    wid = lax.axis_index("s") * 2 + lax.axis_index("c")
    w_base = wid * PER_W
    idx_bufs = (idx0_v, idx1_v)
    drows = (drows0_v, drows1_v)
    trows = (trows0_v, trows1_v)
    dsems = (dsem0, dsem1)
    tsems = (tsem0, tsem1)

    pltpu.sync_copy(table2d_hbm, table_v)

    lane = lax.iota(jnp.int32, 16)
    out_lane = lane * TID_DIM

    def chunk_body(g, carry):
        for b in range(2):
            j = g * 2 + b
            base = w_base + j * CHUNK

            @pl.when(g > 0)
            def _wait_prev_stores():
                pltpu.make_async_copy(
                    drows[b], out_hbm.at[pl.ds(0, DCH)], dsems[b]
                ).wait()
                pltpu.make_async_copy(
                    trows[b], out_hbm.at[pl.ds(0, TCH)], tsems[b]
                ).wait()

            pltpu.sync_copy(idx_hbm.at[pl.ds(base, CHUNK)], idx_bufs[b])

            gcp = pltpu.async_copy(
                table2d_hbm.at[idx_bufs[b].at[pl.ds(0, DCH)]],
                drows[b],
                gsem,
            )

            @plsc.parallel_loop(0, NGRP, unroll=4)
            def grp_body(grp):
                idxv = idx_bufs[b][pl.ds(DCH + grp * 16, 16)]
                row_ids = lane + grp * 16
                for d in range(TID_DIM):
                    dv = jnp.full((16,), d, jnp.int32)
                    val = plsc.load_gather(table_v, [idxv, dv])
                    plsc.store_scatter(trows[b], [row_ids, dv], val)

            gcp.wait()
            pltpu.async_copy(
                drows[b], out_hbm.at[pl.ds(base, DCH)], dsems[b]
            )
            pltpu.async_copy(
                trows[b], out_hbm.at[pl.ds(base + DCH, TCH)], tsems[b]
            )
        return carry

    lax.fori_loop(0, NCHUNK // 2, chunk_body, 0)

    for b in range(2):
        pltpu.make_async_copy(
            drows[b], out_hbm.at[pl.ds(0, DCH)], dsems[b]
        ).wait()
        pltpu.make_async_copy(
            trows[b], out_hbm.at[pl.ds(0, TCH)], tsems[b]
        ).wait()


def kernel(tid, node_emb_in, node_emb_out, tid_table, adp_emb):
    idx = tid.reshape(-1).astype(jnp.int32)
    rows = _gather_kernel(tid_table, idx)
    x_tid = rows.reshape(tid.shape + (TID_DIM,))
    return (node_emb_in, node_emb_out, x_tid, adp_emb)


# speedup vs baseline: 1.9754x; 1.0009x over previous
import functools

import jax
import jax.numpy as jnp
from jax import lax
from jax.experimental import pallas as pl
from jax.experimental.pallas import tpu as pltpu
from jax.experimental.pallas import tpu_sc as plsc

TID_DIM = 32
N_TABLE = 288
N_TOTAL = 64 * 12 * 5000
NW = 32
PER_W = N_TOTAL // NW
CHUNK = 1000
DCH = 664
TCH = CHUNK - DCH
NCHUNK = PER_W // CHUNK
NGRP = TCH // 16
BT_PER_W = (64 * 12) // NW
N_SPLIT = 5000 // CHUNK

_mesh = plsc.VectorSubcoreMesh(core_axis_name="c", subcore_axis_name="s")


@functools.partial(
    pl.kernel,
    mesh=_mesh,
    out_type=jax.ShapeDtypeStruct((64, 12, 5000, TID_DIM), jnp.float32),
    compiler_params=pltpu.CompilerParams(
        use_tc_tiling_on_sc=False, needs_layout_passes=False
    ),
    scratch_types=[
        pltpu.VMEM((N_TABLE, TID_DIM), jnp.float32),
        pltpu.VMEM((CHUNK,), jnp.int32),
        pltpu.VMEM((CHUNK,), jnp.int32),
        pltpu.VMEM((DCH, TID_DIM), jnp.float32),
        pltpu.VMEM((DCH, TID_DIM), jnp.float32),
        pltpu.VMEM((TCH, TID_DIM), jnp.float32),
        pltpu.VMEM((TCH, TID_DIM), jnp.float32),
        pltpu.SemaphoreType.DMA,
        pltpu.SemaphoreType.DMA,
        pltpu.SemaphoreType.DMA,
        pltpu.SemaphoreType.DMA,
        pltpu.SemaphoreType.DMA,
    ],
)
def _gather_kernel(table2d_hbm, idx_hbm, out_hbm,
                   table_v, idx0_v, idx1_v, drows0_v, drows1_v,
                   trows0_v, trows1_v, dsem0, dsem1, tsem0, tsem1, gsem):
    wid = lax.axis_index("s") * 2 + lax.axis_index("c")
    w_base = wid * PER_W
    idx_bufs = (idx0_v, idx1_v)
    drows = (drows0_v, drows1_v)
    trows = (trows0_v, trows1_v)
    dsems = (dsem0, dsem1)
    tsems = (tsem0, tsem1)

    pltpu.sync_copy(table2d_hbm, table_v)

    lane = lax.iota(jnp.int32, 16)
    out_lane = lane * TID_DIM

    def chunk_body(g, carry):
        for b in range(2):
            j = g * 2 + b
            base = w_base + j * CHUNK
            bt = wid * BT_PER_W + j // N_SPLIT
            b_i = bt // 12
            t_i = bt % 12
            n0 = (j % N_SPLIT) * CHUNK

            @pl.when(g > 0)
            def _wait_prev_stores():
                pltpu.make_async_copy(
                    drows[b], out_hbm.at[0, 0, pl.ds(0, DCH)], dsems[b]
                ).wait()
                pltpu.make_async_copy(
                    trows[b], out_hbm.at[0, 0, pl.ds(0, TCH)], tsems[b]
                ).wait()

            pltpu.sync_copy(idx_hbm.at[pl.ds(base, CHUNK)], idx_bufs[b])

            gcp = pltpu.async_copy(
                table2d_hbm.at[idx_bufs[b].at[pl.ds(0, DCH)]],
                drows[b],
                gsem,
            )

            @plsc.parallel_loop(0, NGRP, unroll=4)
            def grp_body(grp):
                idxv = idx_bufs[b][pl.ds(DCH + grp * 16, 16)]
                row_ids = lane + grp * 16
                for d in range(TID_DIM):
                    dv = jnp.full((16,), d, jnp.int32)
                    val = plsc.load_gather(table_v, [idxv, dv])
                    plsc.store_scatter(trows[b], [row_ids, dv], val)

            gcp.wait()
            pltpu.async_copy(
                drows[b],
                out_hbm.at[b_i, t_i, pl.ds(n0, DCH)],
                dsems[b],
            )
            pltpu.async_copy(
                trows[b],
                out_hbm.at[b_i, t_i, pl.ds(n0 + DCH, TCH)],
                tsems[b],
            )
        return carry

    lax.fori_loop(0, NCHUNK // 2, chunk_body, 0)

    for b in range(2):
        pltpu.make_async_copy(
            drows[b], out_hbm.at[0, 0, pl.ds(0, DCH)], dsems[b]
        ).wait()
        pltpu.make_async_copy(
            trows[b], out_hbm.at[0, 0, pl.ds(0, TCH)], tsems[b]
        ).wait()


def kernel(tid, node_emb_in, node_emb_out, tid_table, adp_emb):
    idx = tid.reshape(-1).astype(jnp.int32)
    x_tid = _gather_kernel(tid_table, idx)
    return (node_emb_in, node_emb_out, x_tid, adp_emb)
